# Initial kernel scaffold; baseline (speedup 1.0000x reference)
import jax
import jax.numpy as jnp
from jax.experimental import pallas as pl


def kernel(x, edge_index, edge_attr, edge_vec, W1, b1, W2, b2, P0, P1, P2):
    raise NotImplementedError("write your pallas kernel here")


# speedup vs baseline: 4.6203x; 4.6203x over previous
import functools

import numpy as np
import jax
import jax.numpy as jnp
from jax import lax
from jax.experimental import pallas as pl
from jax.experimental.pallas import tpu as pltpu
from jax.experimental.pallas import tpu_sc as plsc

_NS = 48
_NV = 10
_OUT_W = _NS + 3 * _NV + 5 * _NV


def _repeat_mat(nv, w):
    r = np.zeros((nv, nv * w), np.float32)
    for v in range(nv):
        r[v, v * w:(v + 1) * w] = 1.0
    return r


def _tile_mat(w, nv):
    s = np.zeros((w, nv * w), np.float32)
    for v in range(nv):
        for j in range(w):
            s[j, v * w + j] = 1.0
    return s


_R1 = jnp.asarray(_repeat_mat(_NV, 3))
_R2 = jnp.asarray(_repeat_mat(_NV, 5))
_S1 = jnp.asarray(_tile_mat(3, _NV))
_S2 = jnp.asarray(_tile_mat(5, _NV))


@functools.lru_cache(maxsize=None)
def _make_gather(n, e, ns):
    info = plsc.get_sparse_core_info()
    nc, nsub = info.num_cores, info.num_subcores
    nw = nc * nsub
    epw = e // nw
    k = 1000
    iters = epw // k
    assert epw * nw == e and iters * k == epw and (epw % 8) == 0

    mesh = plsc.VectorSubcoreMesh(core_axis_name="c", subcore_axis_name="s")

    @functools.partial(
        pl.kernel,
        mesh=mesh,
        compiler_params=pltpu.CompilerParams(use_tc_tiling_on_sc=False),
        out_type=(jax.ShapeDtypeStruct((e, ns), jnp.float32),
                  jax.ShapeDtypeStruct((e, ns), jnp.float32)),
        scratch_types=[
            pltpu.VMEM((k,), jnp.int32),
            pltpu.VMEM((k,), jnp.int32),
            pltpu.VMEM((k, ns), jnp.float32),
            pltpu.VMEM((k, ns), jnp.float32),
        ],
    )
    def gather_k(x_hbm, src_hbm, dst_hbm, xs_hbm, xd_hbm, si_v, di_v, sr_v, dr_v):
        wid = lax.axis_index("s") * nc + lax.axis_index("c")
        base = wid * epw

        def step(i, carry):
            off = base + i * k
            pltpu.sync_copy(src_hbm.at[pl.ds(off, k)], si_v)
            pltpu.sync_copy(dst_hbm.at[pl.ds(off, k)], di_v)
            pltpu.sync_copy(x_hbm.at[si_v], sr_v)
            pltpu.sync_copy(sr_v, xs_hbm.at[pl.ds(off, k)])
            pltpu.sync_copy(x_hbm.at[di_v], dr_v)
            pltpu.sync_copy(dr_v, xd_hbm.at[pl.ds(off, k)])
            return carry

        lax.fori_loop(0, iters, step, 0)

    return gather_k


@functools.lru_cache(maxsize=None)
def _make_messages(e, ns, nv):
    be = 6400
    grid = e // be
    assert grid * be == e and be % 128 == 0

    def body(ea, xs, xd, evt, w1a, w1b, w1c, b1, w2a, w2b, w2c, b2a, b2b, b2c,
             p0, p1, p2, r1, r2, s1, s2, out):
        f32 = jnp.float32
        ea_b, xs_b, xd_b = ea[...], xs[...], xd[...]
        h = jnp.dot(ea_b, w1a[...], preferred_element_type=f32)
        h += jnp.dot(xs_b, w1b[...], preferred_element_type=f32)
        h += jnp.dot(xd_b, w1c[...], preferred_element_type=f32)
        h = jax.nn.relu(h + b1[...])
        g0 = jnp.dot(h, w2a[...], preferred_element_type=f32) + b2a[...]
        g1 = jnp.dot(h, w2b[...], preferred_element_type=f32) + b2b[...]
        g2 = jnp.dot(h, w2c[...], preferred_element_type=f32) + b2c[...]
        t0 = g0 * jnp.dot(xs_b, p0[...], preferred_element_type=f32)
        t1 = g1 * jnp.dot(xs_b, p1[...], preferred_element_type=f32)
        t2 = g2 * jnp.dot(xs_b, p2[...], preferred_element_type=f32)
        ev = evt[...]
        vx, vy, vz = ev[0:1, :], ev[1:2, :], ev[2:3, :]
        inv = 1.0 / (jnp.sqrt(vx * vx + vy * vy + vz * vz) + 1e-8)
        nx, ny, nz = vx * inv, vy * inv, vz * inv
        sq3, sq15 = np.sqrt(3.0), np.sqrt(15.0)
        sh1r = jnp.concatenate([nx, ny, nz], axis=0) * sq3
        sh2r = jnp.concatenate([
            nx * ny,
            ny * nz,
            (3.0 * nz * nz - 1.0) / (2.0 * sq3),
            nz * nx,
            (nx * nx - ny * ny) / 2.0,
        ], axis=0) * sq15
        she1 = lax.dot_general(sh1r, s1[...], (((0,), (0,)), ((), ())),
                               preferred_element_type=f32)
        she2 = lax.dot_general(sh2r, s2[...], (((0,), (0,)), ((), ())),
                               preferred_element_type=f32)
        m1 = jnp.dot(t1, r1[...], preferred_element_type=f32) * she1
        m2 = jnp.dot(t2, r2[...], preferred_element_type=f32) * she2
        out[...] = jnp.concatenate([t0, m1, m2], axis=1)

    def full(shape):
        return pl.BlockSpec(shape, lambda i: (0, 0))

    return pl.pallas_call(
        body,
        grid=(grid,),
        in_specs=[
            pl.BlockSpec((be, ns), lambda i: (i, 0)),
            pl.BlockSpec((be, ns), lambda i: (i, 0)),
            pl.BlockSpec((be, ns), lambda i: (i, 0)),
            pl.BlockSpec((3, be), lambda i: (0, i)),
            full((ns, ns)), full((ns, ns)), full((ns, ns)), full((1, ns)),
            full((ns, ns)), full((ns, nv)), full((ns, nv)),
            full((1, ns)), full((1, nv)), full((1, nv)),
            full((ns, ns)), full((ns, nv)), full((ns, nv)),
            full((nv, 3 * nv)), full((nv, 5 * nv)),
            full((3, 3 * nv)), full((5, 5 * nv)),
        ],
        out_specs=pl.BlockSpec((be, _OUT_W), lambda i: (i, 0)),
        out_shape=jax.ShapeDtypeStruct((e, _OUT_W), jnp.float32),
    )


@functools.lru_cache(maxsize=None)
def _make_scatter(n, e):
    info = plsc.get_sparse_core_info()
    nc, nsub = info.num_cores, info.num_subcores
    cw = _OUT_W // 8
    rpt = n // nsub
    rco = 625
    k = 1000
    ept = e // nsub
    iters = ept // k
    dco = 5000
    assert rpt * nsub == n and iters * k == ept and (rpt % rco) == 0
    assert (n % dco) == 0 and (dco % 8) == 0

    mesh = plsc.VectorSubcoreMesh(core_axis_name="c", subcore_axis_name="s")

    @functools.partial(
        pl.kernel,
        mesh=mesh,
        compiler_params=pltpu.CompilerParams(use_tc_tiling_on_sc=False),
        out_type=(jax.ShapeDtypeStruct((n, _OUT_W), jnp.float32),
                  jax.ShapeDtypeStruct((n,), jnp.float32)),
        scratch_types=[
            pltpu.VMEM((k,), jnp.int32),
            pltpu.VMEM((k, cw), jnp.float32),
            pltpu.VMEM((k,), jnp.float32),
            pltpu.VMEM((rco, cw), jnp.float32),
            pltpu.VMEM((dco,), jnp.float32),
            pltpu.VMEM_SHARED((n, cw), jnp.float32),
            pltpu.VMEM_SHARED((n,), jnp.float32),
        ],
    )
    def scatter_k(m_hbm, dst_hbm, z2_hbm, z1_hbm, ones_hbm, osum_hbm, deg_hbm,
                  idx_v, val_v, ones_v, out_v, deg_v, acc_sp, deg_sp):
        cid = lax.axis_index("c")
        sid = lax.axis_index("s")
        r0 = sid * rpt

        def chunk_work(c, do_deg):
            col = c * cw
            plsc.subcore_barrier()

            @pl.when(sid == 0)
            def _zero():
                pltpu.sync_copy(z2_hbm, acc_sp)

            if do_deg:
                @pl.when(sid == 1)
                def _zero_deg():
                    pltpu.sync_copy(z1_hbm, deg_sp)
                pltpu.sync_copy(ones_hbm, ones_v)

            plsc.subcore_barrier()

            def step(i, carry):
                off = sid * ept + i * k
                pltpu.sync_copy(dst_hbm.at[pl.ds(off, k)], idx_v)
                pltpu.sync_copy(m_hbm.at[pl.ds(off, k), pl.ds(col, cw)], val_v)
                pltpu.sync_copy(val_v, acc_sp.at[idx_v], add=True)
                if do_deg:
                    pltpu.sync_copy(ones_v, deg_sp.at[idx_v], add=True)
                return carry

            lax.fori_loop(0, iters, step, 0)
            plsc.subcore_barrier()
            for j in range(rpt // rco):
                rr = r0 + j * rco
                pltpu.sync_copy(acc_sp.at[pl.ds(rr, rco)], out_v)
                pltpu.sync_copy(out_v, osum_hbm.at[pl.ds(rr, rco), pl.ds(col, cw)])
            if do_deg:
                @pl.when(sid == 0)
                def _deg_out():
                    def dstep(j, carry):
                        doff = j * dco
                        pltpu.sync_copy(deg_sp.at[pl.ds(doff, dco)], deg_v)
                        pltpu.sync_copy(deg_v, deg_hbm.at[pl.ds(doff, dco)])
                        return carry
                    lax.fori_loop(0, n // dco, dstep, 0)

        @pl.when(cid == 0)
        def _sc0():
            for t in range(4):
                chunk_work(t, t == 0)

        @pl.when(cid == 1)
        def _sc1():
            for t in range(4, 8):
                chunk_work(t, False)

    return scatter_k


@functools.lru_cache(maxsize=None)
def _make_divide(n):
    bn = 2000
    grid = n // bn
    assert grid * bn == n

    def body(osum, deg, out):
        out[...] = osum[...] / jnp.maximum(deg[...], 1.0)

    return pl.pallas_call(
        body,
        grid=(grid,),
        in_specs=[
            pl.BlockSpec((bn, _OUT_W), lambda i: (i, 0)),
            pl.BlockSpec((bn, 1), lambda i: (i, 0)),
        ],
        out_specs=pl.BlockSpec((bn, _OUT_W), lambda i: (i, 0)),
        out_shape=jax.ShapeDtypeStruct((n, _OUT_W), jnp.float32),
    )


def kernel(x, edge_index, edge_attr, edge_vec, W1, b1, W2, b2, P0, P1, P2):
    n, ns = x.shape
    e = edge_index.shape[1]
    nv = P1.shape[1]
    src = edge_index[0]
    dst = edge_index[1]

    xs, xd = _make_gather(n, e, ns)(x, src, dst)

    evt = edge_vec.T
    w1a, w1b, w1c = W1[:ns], W1[ns:2 * ns], W1[2 * ns:]
    w2a, w2b, w2c = W2[:, :ns], W2[:, ns:ns + nv], W2[:, ns + nv:]
    b1r = b1.reshape(1, ns)
    b2a, b2b, b2c = (b2[:ns].reshape(1, ns), b2[ns:ns + nv].reshape(1, nv),
                     b2[ns + nv:].reshape(1, nv))
    m = _make_messages(e, ns, nv)(
        edge_attr, xs, xd, evt, w1a, w1b, w1c, b1r, w2a, w2b, w2c,
        b2a, b2b, b2c, P0, P1, P2, _R1, _R2, _S1, _S2)

    z2 = jnp.zeros((n, _OUT_W // 8), jnp.float32)
    z1 = jnp.zeros((n,), jnp.float32)
    ones = jnp.ones((1000,), jnp.float32)
    osum, deg = _make_scatter(n, e)(m, dst, z2, z1, ones)

    return _make_divide(n)(osum, deg.reshape(n, 1))


# speedup vs baseline: 5.0134x; 1.0851x over previous
import functools

import numpy as np
import jax
import jax.numpy as jnp
from jax import lax
from jax.experimental import pallas as pl
from jax.experimental.pallas import tpu as pltpu
from jax.experimental.pallas import tpu_sc as plsc

_NS = 48
_NV = 10
_OUT_W = _NS + 3 * _NV + 5 * _NV


def _repeat_mat(nv, w):
    r = np.zeros((nv, nv * w), np.float32)
    for v in range(nv):
        r[v, v * w:(v + 1) * w] = 1.0
    return r


def _tile_mat(w, nv):
    s = np.zeros((w, nv * w), np.float32)
    for v in range(nv):
        for j in range(w):
            s[j, v * w + j] = 1.0
    return s


def _blockdiag(a, b):
    out = np.zeros((a.shape[0] + b.shape[0], a.shape[1] + b.shape[1]), np.float32)
    out[:a.shape[0], :a.shape[1]] = a
    out[a.shape[0]:, a.shape[1]:] = b
    return out


_RCAT = jnp.asarray(_blockdiag(_repeat_mat(_NV, 3), _repeat_mat(_NV, 5)))
_SCAT = jnp.asarray(_blockdiag(_tile_mat(3, _NV), _tile_mat(5, _NV)))


@functools.lru_cache(maxsize=None)
def _make_gather(n, e, ns):
    info = plsc.get_sparse_core_info()
    nc, nsub = info.num_cores, info.num_subcores
    nw = nc * nsub
    epw = e // nw
    k = 1000
    iters = epw // k
    assert epw * nw == e and iters * k == epw and (epw % 8) == 0

    mesh = plsc.VectorSubcoreMesh(core_axis_name="c", subcore_axis_name="s")

    @functools.partial(
        pl.kernel,
        mesh=mesh,
        compiler_params=pltpu.CompilerParams(use_tc_tiling_on_sc=False),
        out_type=(jax.ShapeDtypeStruct((e, ns), jnp.float32),
                  jax.ShapeDtypeStruct((e, ns), jnp.float32)),
        scratch_types=[
            pltpu.VMEM((k,), jnp.int32),
            pltpu.VMEM((k,), jnp.int32),
            pltpu.VMEM((k, ns), jnp.float32),
            pltpu.VMEM((k, ns), jnp.float32),
        ],
    )
    def gather_k(x_hbm, src_hbm, dst_hbm, xs_hbm, xd_hbm, si_v, di_v, sr_v, dr_v):
        wid = lax.axis_index("s") * nc + lax.axis_index("c")
        base = wid * epw

        def step(i, carry):
            off = base + i * k
            pltpu.sync_copy(src_hbm.at[pl.ds(off, k)], si_v)
            pltpu.sync_copy(dst_hbm.at[pl.ds(off, k)], di_v)
            pltpu.sync_copy(x_hbm.at[si_v], sr_v)
            pltpu.sync_copy(sr_v, xs_hbm.at[pl.ds(off, k)])
            pltpu.sync_copy(x_hbm.at[di_v], dr_v)
            pltpu.sync_copy(dr_v, xd_hbm.at[pl.ds(off, k)])
            return carry

        lax.fori_loop(0, iters, step, 0)

    return gather_k


@functools.lru_cache(maxsize=None)
def _make_messages(e, ns, nv):
    be = 6400
    grid = e // be
    assert grid * be == e and be % 128 == 0

    ng = ns + 2 * nv

    def body(ea, xs, xd, evt, b1big, w2, b1, b2, rcat, scat, out):
        f32 = jnp.float32
        cat = jnp.concatenate([ea[...], xs[...], xd[...]], axis=1)
        hp = jnp.dot(cat, b1big[...], preferred_element_type=f32)
        h = jax.nn.relu(hp[:, :ns] + b1[...])
        p = hp[:, ns:]
        g = jnp.dot(h, w2[...], preferred_element_type=f32) + b2[...]
        t = g * p
        a12 = jnp.dot(t[:, ns:], rcat[...], preferred_element_type=f32)
        ev = evt[...]
        vx, vy, vz = ev[0:1, :], ev[1:2, :], ev[2:3, :]
        inv = 1.0 / (jnp.sqrt(vx * vx + vy * vy + vz * vz) + 1e-8)
        nx, ny, nz = vx * inv, vy * inv, vz * inv
        sq3, sq15 = np.sqrt(3.0), np.sqrt(15.0)
        shr = jnp.concatenate([
            nx * sq3, ny * sq3, nz * sq3,
            nx * ny * sq15,
            ny * nz * sq15,
            (3.0 * nz * nz - 1.0) / (2.0 * sq3) * sq15,
            nz * nx * sq15,
            (nx * nx - ny * ny) / 2.0 * sq15,
        ], axis=0)
        she = lax.dot_general(shr, scat[...], (((0,), (0,)), ((), ())),
                              preferred_element_type=f32)
        out[...] = jnp.concatenate([t[:, :ns], a12 * she], axis=1)

    def full(shape):
        return pl.BlockSpec(shape, lambda i: (0, 0))

    return pl.pallas_call(
        body,
        grid=(grid,),
        in_specs=[
            pl.BlockSpec((be, ns), lambda i: (i, 0)),
            pl.BlockSpec((be, ns), lambda i: (i, 0)),
            pl.BlockSpec((be, ns), lambda i: (i, 0)),
            pl.BlockSpec((3, be), lambda i: (0, i)),
            full((3 * ns, ns + ng)),
            full((ns, ng)),
            full((1, ns)), full((1, ng)),
            full((2 * nv, 8 * nv)),
            full((8, 8 * nv)),
        ],
        out_specs=pl.BlockSpec((be, _OUT_W), lambda i: (i, 0)),
        out_shape=jax.ShapeDtypeStruct((e, _OUT_W), jnp.float32),
    )


@functools.lru_cache(maxsize=None)
def _make_scatter(n, e):
    info = plsc.get_sparse_core_info()
    nc, nsub = info.num_cores, info.num_subcores
    cw = _OUT_W // 8
    rpt = n // nsub
    rco = 625
    k = 1000
    ept = e // nsub
    iters = ept // k
    dco = 5000
    assert rpt * nsub == n and iters * k == ept and (rpt % rco) == 0
    assert (n % dco) == 0 and (dco % 8) == 0

    mesh = plsc.VectorSubcoreMesh(core_axis_name="c", subcore_axis_name="s")

    @functools.partial(
        pl.kernel,
        mesh=mesh,
        compiler_params=pltpu.CompilerParams(use_tc_tiling_on_sc=False),
        out_type=(jax.ShapeDtypeStruct((n, _OUT_W), jnp.float32),
                  jax.ShapeDtypeStruct((n,), jnp.float32)),
        scratch_types=[
            pltpu.VMEM((k,), jnp.int32),
            pltpu.VMEM((k, cw), jnp.float32),
            pltpu.VMEM((k,), jnp.float32),
            pltpu.VMEM((rco, cw), jnp.float32),
            pltpu.VMEM((dco,), jnp.float32),
            pltpu.VMEM_SHARED((n, cw), jnp.float32),
            pltpu.VMEM_SHARED((n,), jnp.float32),
        ],
    )
    def scatter_k(m_hbm, dst_hbm, z2_hbm, z1_hbm, ones_hbm, osum_hbm, deg_hbm,
                  idx_v, val_v, ones_v, out_v, deg_v, acc_sp, deg_sp):
        cid = lax.axis_index("c")
        sid = lax.axis_index("s")
        r0 = sid * rpt

        def chunk_work(c, do_deg):
            col = c * cw
            plsc.subcore_barrier()

            @pl.when(sid == 0)
            def _zero():
                pltpu.sync_copy(z2_hbm, acc_sp)

            if do_deg:
                @pl.when(sid == 1)
                def _zero_deg():
                    pltpu.sync_copy(z1_hbm, deg_sp)
                pltpu.sync_copy(ones_hbm, ones_v)

            plsc.subcore_barrier()

            def step(i, carry):
                off = sid * ept + i * k
                pltpu.sync_copy(dst_hbm.at[pl.ds(off, k)], idx_v)
                pltpu.sync_copy(m_hbm.at[pl.ds(off, k), pl.ds(col, cw)], val_v)
                pltpu.sync_copy(val_v, acc_sp.at[idx_v], add=True)
                if do_deg:
                    pltpu.sync_copy(ones_v, deg_sp.at[idx_v], add=True)
                return carry

            lax.fori_loop(0, iters, step, 0)
            plsc.subcore_barrier()
            for j in range(rpt // rco):
                rr = r0 + j * rco
                pltpu.sync_copy(acc_sp.at[pl.ds(rr, rco)], out_v)
                pltpu.sync_copy(out_v, osum_hbm.at[pl.ds(rr, rco), pl.ds(col, cw)])
            if do_deg:
                @pl.when(sid == 0)
                def _deg_out():
                    def dstep(j, carry):
                        doff = j * dco
                        pltpu.sync_copy(deg_sp.at[pl.ds(doff, dco)], deg_v)
                        pltpu.sync_copy(deg_v, deg_hbm.at[pl.ds(doff, dco)])
                        return carry
                    lax.fori_loop(0, n // dco, dstep, 0)

        @pl.when(cid == 0)
        def _sc0():
            for t in range(4):
                chunk_work(t, t == 0)

        @pl.when(cid == 1)
        def _sc1():
            for t in range(4, 8):
                chunk_work(t, False)

    return scatter_k


@functools.lru_cache(maxsize=None)
def _make_divide(n):
    bn = 2000
    grid = n // bn
    assert grid * bn == n

    def body(osum, deg, out):
        out[...] = osum[...] / jnp.maximum(deg[...], 1.0)

    return pl.pallas_call(
        body,
        grid=(grid,),
        in_specs=[
            pl.BlockSpec((bn, _OUT_W), lambda i: (i, 0)),
            pl.BlockSpec((bn, 1), lambda i: (i, 0)),
        ],
        out_specs=pl.BlockSpec((bn, _OUT_W), lambda i: (i, 0)),
        out_shape=jax.ShapeDtypeStruct((n, _OUT_W), jnp.float32),
    )


def kernel(x, edge_index, edge_attr, edge_vec, W1, b1, W2, b2, P0, P1, P2):
    n, ns = x.shape
    e = edge_index.shape[1]
    nv = P1.shape[1]
    src = edge_index[0]
    dst = edge_index[1]

    xs, xd = _make_gather(n, e, ns)(x, src, dst)

    evt = edge_vec.T
    ng = ns + 2 * nv
    pp = jnp.concatenate([P0, P1, P2], axis=1)
    zpad = jnp.zeros((ns, ng), jnp.float32)
    b1big = jnp.concatenate([
        jnp.concatenate([W1[:ns], zpad], axis=1),
        jnp.concatenate([W1[ns:2 * ns], pp], axis=1),
        jnp.concatenate([W1[2 * ns:], zpad], axis=1),
    ], axis=0)
    m = _make_messages(e, ns, nv)(
        edge_attr, xs, xd, evt, b1big, W2, b1.reshape(1, ns),
        b2.reshape(1, ng), _RCAT, _SCAT)

    z2 = jnp.zeros((n, _OUT_W // 8), jnp.float32)
    z1 = jnp.zeros((n,), jnp.float32)
    ones = jnp.ones((1000,), jnp.float32)
    osum, deg = _make_scatter(n, e)(m, dst, z2, z1, ones)

    return _make_divide(n)(osum, deg.reshape(n, 1))


# speedup vs baseline: 6.9864x; 1.3935x over previous
import functools

import numpy as np
import jax
import jax.numpy as jnp
from jax import lax
from jax.experimental import pallas as pl
from jax.experimental.pallas import tpu as pltpu
from jax.experimental.pallas import tpu_sc as plsc

_NS = 48
_NV = 10
_OUT_W = _NS + 3 * _NV + 5 * _NV


def _repeat_mat(nv, w):
    r = np.zeros((nv, nv * w), np.float32)
    for v in range(nv):
        r[v, v * w:(v + 1) * w] = 1.0
    return r


def _tile_mat(w, nv):
    s = np.zeros((w, nv * w), np.float32)
    for v in range(nv):
        for j in range(w):
            s[j, v * w + j] = 1.0
    return s


def _blockdiag(a, b):
    out = np.zeros((a.shape[0] + b.shape[0], a.shape[1] + b.shape[1]), np.float32)
    out[:a.shape[0], :a.shape[1]] = a
    out[a.shape[0]:, a.shape[1]:] = b
    return out


_RCAT = jnp.asarray(_blockdiag(_repeat_mat(_NV, 3), _repeat_mat(_NV, 5)))
_SCAT = jnp.asarray(_blockdiag(_tile_mat(3, _NV), _tile_mat(5, _NV)))


@functools.lru_cache(maxsize=None)
def _make_gather(n, e, ns):
    info = plsc.get_sparse_core_info()
    nc, nsub = info.num_cores, info.num_subcores
    nw = nc * nsub
    epw = e // nw
    k = 1000
    iters = epw // k
    assert epw * nw == e and iters * k == epw and (epw % 8) == 0

    mesh = plsc.VectorSubcoreMesh(core_axis_name="c", subcore_axis_name="s")

    @functools.partial(
        pl.kernel,
        mesh=mesh,
        compiler_params=pltpu.CompilerParams(use_tc_tiling_on_sc=False),
        out_type=jax.ShapeDtypeStruct((e, 128), jnp.float32),
        scratch_types=[
            pltpu.VMEM((k,), jnp.int32),
            pltpu.VMEM((k,), jnp.int32),
            pltpu.VMEM((k, ns), jnp.float32),
            pltpu.VMEM((k, ns), jnp.float32),
        ],
    )
    def gather_k(x_hbm, src_hbm, dst_hbm, xsd_hbm, si_v, di_v, sr_v, dr_v):
        wid = lax.axis_index("s") * nc + lax.axis_index("c")
        base = wid * epw

        def step(i, carry):
            off = base + i * k
            pltpu.sync_copy(src_hbm.at[pl.ds(off, k)], si_v)
            pltpu.sync_copy(dst_hbm.at[pl.ds(off, k)], di_v)
            pltpu.sync_copy(x_hbm.at[si_v], sr_v)
            pltpu.sync_copy(sr_v, xsd_hbm.at[pl.ds(off, k), pl.ds(0, ns)])
            pltpu.sync_copy(x_hbm.at[di_v], dr_v)
            pltpu.sync_copy(dr_v, xsd_hbm.at[pl.ds(off, k), pl.ds(ns, ns)])
            return carry

        lax.fori_loop(0, iters, step, 0)

    return gather_k


@functools.lru_cache(maxsize=None)
def _make_messages(e, ns, nv):
    be = 6400
    grid = e // be
    assert grid * be == e and be % 128 == 0

    ng = ns + 2 * nv

    def body(ea, xsd, evt, b1big, w2, b1, b2, rcat, scat, out):
        f32 = jnp.float32
        xsd_b = xsd[...]
        cat = jnp.concatenate([ea[...], xsd_b[:, :ns], xsd_b[:, ns:2 * ns]],
                              axis=1)
        hp = jnp.dot(cat, b1big[...], preferred_element_type=f32)
        h = jax.nn.relu(hp[:, :ns] + b1[...])
        p = hp[:, ns:]
        g = jnp.dot(h, w2[...], preferred_element_type=f32) + b2[...]
        t = g * p
        a12 = jnp.dot(t[:, ns:], rcat[...], preferred_element_type=f32)
        ev = evt[...]
        vx, vy, vz = ev[0:1, :], ev[1:2, :], ev[2:3, :]
        inv = 1.0 / (jnp.sqrt(vx * vx + vy * vy + vz * vz) + 1e-8)
        nx, ny, nz = vx * inv, vy * inv, vz * inv
        sq3, sq15 = np.sqrt(3.0), np.sqrt(15.0)
        shr = jnp.concatenate([
            nx * sq3, ny * sq3, nz * sq3,
            nx * ny * sq15,
            ny * nz * sq15,
            (3.0 * nz * nz - 1.0) / (2.0 * sq3) * sq15,
            nz * nx * sq15,
            (nx * nx - ny * ny) / 2.0 * sq15,
        ], axis=0)
        she = lax.dot_general(shr, scat[...], (((0,), (0,)), ((), ())),
                              preferred_element_type=f32)
        out[...] = jnp.concatenate([t[:, :ns], a12 * she], axis=1)

    def full(shape):
        return pl.BlockSpec(shape, lambda i: (0, 0))

    return pl.pallas_call(
        body,
        grid=(grid,),
        in_specs=[
            pl.BlockSpec((be, ns), lambda i: (i, 0)),
            pl.BlockSpec((be, 128), lambda i: (i, 0)),
            pl.BlockSpec((3, be), lambda i: (0, i)),
            full((3 * ns, ns + ng)),
            full((ns, ng)),
            full((1, ns)), full((1, ng)),
            full((2 * nv, 8 * nv)),
            full((8, 8 * nv)),
        ],
        out_specs=pl.BlockSpec((be, _OUT_W), lambda i: (i, 0)),
        out_shape=jax.ShapeDtypeStruct((e, _OUT_W), jnp.float32),
    )


@functools.lru_cache(maxsize=None)
def _make_scatter(n, e):
    info = plsc.get_sparse_core_info()
    nc, nsub = info.num_cores, info.num_subcores
    cw = _OUT_W // 8
    rpt = n // nsub
    rco = 625
    k = 1000
    ept = e // nsub
    iters = ept // k
    dco = 5000
    assert rpt * nsub == n and iters * k == ept and (rpt % rco) == 0
    assert (n % dco) == 0 and (dco % 8) == 0

    mesh = plsc.VectorSubcoreMesh(core_axis_name="c", subcore_axis_name="s")

    @functools.partial(
        pl.kernel,
        mesh=mesh,
        compiler_params=pltpu.CompilerParams(use_tc_tiling_on_sc=False),
        out_type=(jax.ShapeDtypeStruct((n, _OUT_W), jnp.float32),
                  jax.ShapeDtypeStruct((n,), jnp.float32)),
        scratch_types=[
            pltpu.VMEM((k,), jnp.int32),
            pltpu.VMEM((k, cw), jnp.float32),
            pltpu.VMEM((k,), jnp.float32),
            pltpu.VMEM((rco, cw), jnp.float32),
            pltpu.VMEM((dco,), jnp.float32),
            pltpu.VMEM_SHARED((n, cw), jnp.float32),
            pltpu.VMEM_SHARED((n,), jnp.float32),
        ],
    )
    def scatter_k(m_hbm, dst_hbm, z2_hbm, z1_hbm, ones_hbm, osum_hbm, deg_hbm,
                  idx_v, val_v, ones_v, out_v, deg_v, acc_sp, deg_sp):
        cid = lax.axis_index("c")
        sid = lax.axis_index("s")
        r0 = sid * rpt

        def chunk_work(c, do_deg):
            col = c * cw
            plsc.subcore_barrier()

            @pl.when(sid == 0)
            def _zero():
                pltpu.sync_copy(z2_hbm, acc_sp)

            if do_deg:
                @pl.when(sid == 1)
                def _zero_deg():
                    pltpu.sync_copy(z1_hbm, deg_sp)
                pltpu.sync_copy(ones_hbm, ones_v)

            plsc.subcore_barrier()

            def step(i, carry):
                off = sid * ept + i * k
                pltpu.sync_copy(dst_hbm.at[pl.ds(off, k)], idx_v)
                pltpu.sync_copy(m_hbm.at[pl.ds(off, k), pl.ds(col, cw)], val_v)
                pltpu.sync_copy(val_v, acc_sp.at[idx_v], add=True)
                if do_deg:
                    pltpu.sync_copy(ones_v, deg_sp.at[idx_v], add=True)
                return carry

            lax.fori_loop(0, iters, step, 0)
            plsc.subcore_barrier()
            for j in range(rpt // rco):
                rr = r0 + j * rco
                pltpu.sync_copy(acc_sp.at[pl.ds(rr, rco)], out_v)
                pltpu.sync_copy(out_v, osum_hbm.at[pl.ds(rr, rco), pl.ds(col, cw)])
            if do_deg:
                @pl.when(sid == 0)
                def _deg_out():
                    def dstep(j, carry):
                        doff = j * dco
                        pltpu.sync_copy(deg_sp.at[pl.ds(doff, dco)], deg_v)
                        pltpu.sync_copy(deg_v, deg_hbm.at[pl.ds(doff, dco)])
                        return carry
                    lax.fori_loop(0, n // dco, dstep, 0)

        @pl.when(cid == 0)
        def _sc0():
            for t in range(4):
                chunk_work(t, t == 0)

        @pl.when(cid == 1)
        def _sc1():
            for t in range(4, 8):
                chunk_work(t, False)

    return scatter_k


@functools.lru_cache(maxsize=None)
def _make_divide(n):
    bn = 2000
    grid = n // bn
    assert grid * bn == n

    def body(osum, deg, out):
        out[...] = osum[...] / jnp.maximum(deg[...], 1.0)

    return pl.pallas_call(
        body,
        grid=(grid,),
        in_specs=[
            pl.BlockSpec((bn, _OUT_W), lambda i: (i, 0)),
            pl.BlockSpec((bn, 1), lambda i: (i, 0)),
        ],
        out_specs=pl.BlockSpec((bn, _OUT_W), lambda i: (i, 0)),
        out_shape=jax.ShapeDtypeStruct((n, _OUT_W), jnp.float32),
    )


def kernel(x, edge_index, edge_attr, edge_vec, W1, b1, W2, b2, P0, P1, P2):
    n, ns = x.shape
    e = edge_index.shape[1]
    nv = P1.shape[1]
    src = edge_index[0]
    dst = edge_index[1]

    xsd = _make_gather(n, e, ns)(x, src, dst)

    evt = edge_vec.T
    ng = ns + 2 * nv
    pp = jnp.concatenate([P0, P1, P2], axis=1)
    zpad = jnp.zeros((ns, ng), jnp.float32)
    b1big = jnp.concatenate([
        jnp.concatenate([W1[:ns], zpad], axis=1),
        jnp.concatenate([W1[ns:2 * ns], pp], axis=1),
        jnp.concatenate([W1[2 * ns:], zpad], axis=1),
    ], axis=0)
    m = _make_messages(e, ns, nv)(
        edge_attr, xsd, evt, b1big, W2, b1.reshape(1, ns),
        b2.reshape(1, ng), _RCAT, _SCAT)

    z2 = jnp.zeros((n, _OUT_W // 8), jnp.float32)
    z1 = jnp.zeros((n,), jnp.float32)
    ones = jnp.ones((1000,), jnp.float32)
    osum, deg = _make_scatter(n, e)(m, dst, z2, z1, ones)

    return _make_divide(n)(osum, deg.reshape(n, 1))


# speedup vs baseline: 8.6085x; 1.2322x over previous
import functools

import numpy as np
import jax
import jax.numpy as jnp
from jax import lax
from jax.experimental import pallas as pl
from jax.experimental.pallas import tpu as pltpu
from jax.experimental.pallas import tpu_sc as plsc

_NS = 48
_NV = 10
_OUT_W = _NS + 3 * _NV + 5 * _NV


def _repeat_mat(nv, w):
    r = np.zeros((nv, nv * w), np.float32)
    for v in range(nv):
        r[v, v * w:(v + 1) * w] = 1.0
    return r


def _tile_mat(w, nv):
    s = np.zeros((w, nv * w), np.float32)
    for v in range(nv):
        for j in range(w):
            s[j, v * w + j] = 1.0
    return s


def _blockdiag(a, b):
    out = np.zeros((a.shape[0] + b.shape[0], a.shape[1] + b.shape[1]), np.float32)
    out[:a.shape[0], :a.shape[1]] = a
    out[a.shape[0]:, a.shape[1]:] = b
    return out


_RCAT = _blockdiag(_repeat_mat(_NV, 3), _repeat_mat(_NV, 5))
_SCAT = _blockdiag(_tile_mat(3, _NV), _tile_mat(5, _NV))


@functools.lru_cache(maxsize=None)
def _make_gather(n, e, ns):
    info = plsc.get_sparse_core_info()
    nc, nsub = info.num_cores, info.num_subcores
    nw = nc * nsub
    epw = e // nw
    k = 1000
    iters = epw // k
    assert epw * nw == e and iters * k == epw and (epw % 8) == 0

    mesh = plsc.VectorSubcoreMesh(core_axis_name="c", subcore_axis_name="s")

    @functools.partial(
        pl.kernel,
        mesh=mesh,
        compiler_params=pltpu.CompilerParams(use_tc_tiling_on_sc=False),
        out_type=jax.ShapeDtypeStruct((e, 128), jnp.float32),
        scratch_types=[
            pltpu.VMEM((k,), jnp.int32),
            pltpu.VMEM((k,), jnp.int32),
            pltpu.VMEM((k, ns), jnp.float32),
            pltpu.VMEM((k, ns), jnp.float32),
            pltpu.SemaphoreType.DMA,
            pltpu.SemaphoreType.DMA,
            pltpu.SemaphoreType.DMA,
        ],
    )
    def gather_k(x_hbm, src_hbm, dst_hbm, xsd_hbm, si_v, di_v, sr_v, dr_v,
                 s_idx, s_g, s_st):
        wid = lax.axis_index("s") * nc + lax.axis_index("c")
        base = wid * epw

        def idx_load(off):
            pltpu.async_copy(src_hbm.at[pl.ds(off, k)], si_v, s_idx)
            pltpu.async_copy(dst_hbm.at[pl.ds(off, k)], di_v, s_idx)

        def idx_wait(off):
            pltpu.make_async_copy(src_hbm.at[pl.ds(off, k)], si_v, s_idx).wait()
            pltpu.make_async_copy(dst_hbm.at[pl.ds(off, k)], di_v, s_idx).wait()

        def st(off):
            pltpu.async_copy(sr_v, xsd_hbm.at[pl.ds(off, k), pl.ds(0, ns)], s_st)
            pltpu.async_copy(dr_v, xsd_hbm.at[pl.ds(off, k), pl.ds(ns, ns)], s_st)

        def st_wait(off):
            pltpu.make_async_copy(
                sr_v, xsd_hbm.at[pl.ds(off, k), pl.ds(0, ns)], s_st).wait()
            pltpu.make_async_copy(
                dr_v, xsd_hbm.at[pl.ds(off, k), pl.ds(ns, ns)], s_st).wait()

        idx_load(base)

        def step(i, carry):
            off = base + i * k

            @pl.when(i > 0)
            def _():
                st_wait(off - k)

            idx_wait(off)
            pltpu.async_copy(x_hbm.at[si_v], sr_v, s_g)
            pltpu.async_copy(x_hbm.at[di_v], dr_v, s_g)
            pltpu.make_async_copy(x_hbm.at[si_v], sr_v, s_g).wait()
            pltpu.make_async_copy(x_hbm.at[di_v], dr_v, s_g).wait()

            @pl.when(i + 1 < iters)
            def _():
                idx_load(off + k)

            st(off)
            return carry

        lax.fori_loop(0, iters, step, 0)
        st_wait(base + (iters - 1) * k)

    return gather_k


@functools.lru_cache(maxsize=None)
def _make_messages(e, ns, nv):
    be = 6400
    grid = e // be
    assert grid * be == e and be % 128 == 0

    ng = ns + 2 * nv

    def body(ea, xsd, evt, b1big, w2, b1, b2, rcat, scat, out):
        f32 = jnp.float32
        xsd_b = xsd[...]
        cat = jnp.concatenate([ea[...], xsd_b[:, :ns], xsd_b[:, ns:2 * ns]],
                              axis=1)
        hp = jnp.dot(cat, b1big[...], preferred_element_type=f32)
        h = jax.nn.relu(hp[:, :ns] + b1[...])
        p = hp[:, ns:]
        g = jnp.dot(h, w2[...], preferred_element_type=f32) + b2[...]
        t = g * p
        a12 = jnp.dot(t[:, ns:], rcat[...], preferred_element_type=f32)
        ev = evt[...]
        vx, vy, vz = ev[0:1, :], ev[1:2, :], ev[2:3, :]
        inv = 1.0 / (jnp.sqrt(vx * vx + vy * vy + vz * vz) + 1e-8)
        nx, ny, nz = vx * inv, vy * inv, vz * inv
        sq3, sq15 = np.sqrt(3.0), np.sqrt(15.0)
        shr = jnp.concatenate([
            nx * sq3, ny * sq3, nz * sq3,
            nx * ny * sq15,
            ny * nz * sq15,
            (3.0 * nz * nz - 1.0) / (2.0 * sq3) * sq15,
            nz * nx * sq15,
            (nx * nx - ny * ny) / 2.0 * sq15,
        ], axis=0)
        she = lax.dot_general(shr, scat[...], (((0,), (0,)), ((), ())),
                              preferred_element_type=f32)
        out[...] = jnp.concatenate([t[:, :ns], a12 * she], axis=1)

    def full(shape):
        return pl.BlockSpec(shape, lambda i: (0, 0))

    return pl.pallas_call(
        body,
        grid=(grid,),
        in_specs=[
            pl.BlockSpec((be, ns), lambda i: (i, 0)),
            pl.BlockSpec((be, 128), lambda i: (i, 0)),
            pl.BlockSpec((3, be), lambda i: (0, i)),
            full((3 * ns, ns + ng)),
            full((ns, ng)),
            full((1, ns)), full((1, ng)),
            full((2 * nv, 8 * nv)),
            full((8, 8 * nv)),
        ],
        out_specs=pl.BlockSpec((be, _OUT_W), lambda i: (i, 0)),
        out_shape=jax.ShapeDtypeStruct((e, _OUT_W), jnp.float32),
    )


@functools.lru_cache(maxsize=None)
def _make_scatter(n, e):
    info = plsc.get_sparse_core_info()
    nc, nsub = info.num_cores, info.num_subcores
    cw = _OUT_W // 8
    rpt = n // nsub
    rco = 625
    k = 1000
    ept = e // nsub
    iters = ept // k
    dco = 5000
    assert (iters % 2) == 0
    assert rpt * nsub == n and iters * k == ept and (rpt % rco) == 0
    assert (n % dco) == 0 and (dco % 8) == 0

    mesh = plsc.VectorSubcoreMesh(core_axis_name="c", subcore_axis_name="s")

    @functools.partial(
        pl.kernel,
        mesh=mesh,
        compiler_params=pltpu.CompilerParams(use_tc_tiling_on_sc=False),
        out_type=(jax.ShapeDtypeStruct((n, _OUT_W), jnp.float32),
                  jax.ShapeDtypeStruct((n,), jnp.float32)),
        scratch_types=[
            pltpu.VMEM((k,), jnp.int32),
            pltpu.VMEM((k, cw), jnp.float32),
            pltpu.VMEM((k,), jnp.int32),
            pltpu.VMEM((k, cw), jnp.float32),
            pltpu.VMEM((k,), jnp.float32),
            pltpu.VMEM((rco, cw), jnp.float32),
            pltpu.VMEM((dco,), jnp.float32),
            pltpu.VMEM_SHARED((n, cw), jnp.float32),
            pltpu.VMEM_SHARED((n,), jnp.float32),
            pltpu.SemaphoreType.DMA,
            pltpu.SemaphoreType.DMA,
        ],
    )
    def scatter_k(m_hbm, dst_hbm, z2_hbm, z1_hbm, ones_hbm, osum_hbm, deg_hbm,
                  idx_a, val_a, idx_b, val_b, ones_v, out_v, deg_v,
                  acc_sp, deg_sp, s_a, s_b):
        cid = lax.axis_index("c")
        sid = lax.axis_index("s")
        r0 = sid * rpt
        half = iters // 2

        def chunk_work(c, do_deg):
            col = c * cw
            ebase = sid * ept

            def load(off, idx_v, val_v, sem):
                pltpu.async_copy(dst_hbm.at[pl.ds(off, k)], idx_v, sem)
                pltpu.async_copy(
                    m_hbm.at[pl.ds(off, k), pl.ds(col, cw)], val_v, sem)

            def load_wait(off, idx_v, val_v, sem):
                pltpu.make_async_copy(
                    dst_hbm.at[pl.ds(off, k)], idx_v, sem).wait()
                pltpu.make_async_copy(
                    m_hbm.at[pl.ds(off, k), pl.ds(col, cw)], val_v, sem).wait()

            plsc.subcore_barrier()

            @pl.when(sid == 0)
            def _zero():
                pltpu.sync_copy(z2_hbm, acc_sp)

            if do_deg:
                @pl.when(sid == 1)
                def _zero_deg():
                    pltpu.sync_copy(z1_hbm, deg_sp)
                pltpu.sync_copy(ones_hbm, ones_v)

            load(ebase, idx_a, val_a, s_a)
            plsc.subcore_barrier()

            def step(j, carry):
                offa = ebase + 2 * j * k
                offb = offa + k
                load_wait(offa, idx_a, val_a, s_a)
                load(offb, idx_b, val_b, s_b)
                pltpu.sync_copy(val_a, acc_sp.at[idx_a], add=True)
                if do_deg:
                    pltpu.sync_copy(ones_v, deg_sp.at[idx_a], add=True)

                @pl.when(j + 1 < half)
                def _():
                    load(offa + 2 * k, idx_a, val_a, s_a)

                load_wait(offb, idx_b, val_b, s_b)
                pltpu.sync_copy(val_b, acc_sp.at[idx_b], add=True)
                if do_deg:
                    pltpu.sync_copy(ones_v, deg_sp.at[idx_b], add=True)
                return carry

            lax.fori_loop(0, half, step, 0)
            plsc.subcore_barrier()
            for j in range(rpt // rco):
                rr = r0 + j * rco
                pltpu.sync_copy(acc_sp.at[pl.ds(rr, rco)], out_v)
                pltpu.sync_copy(out_v, osum_hbm.at[pl.ds(rr, rco), pl.ds(col, cw)])
            if do_deg:
                @pl.when(sid == 0)
                def _deg_out():
                    def dstep(j, carry):
                        doff = j * dco
                        pltpu.sync_copy(deg_sp.at[pl.ds(doff, dco)], deg_v)
                        pltpu.sync_copy(deg_v, deg_hbm.at[pl.ds(doff, dco)])
                        return carry
                    lax.fori_loop(0, n // dco, dstep, 0)

        @pl.when(cid == 0)
        def _sc0():
            for t in range(4):
                chunk_work(t, t == 0)

        @pl.when(cid == 1)
        def _sc1():
            for t in range(4, 8):
                chunk_work(t, False)

    return scatter_k


@functools.lru_cache(maxsize=None)
def _make_divide(n):
    bn = 2000
    grid = n // bn
    assert grid * bn == n

    def body(osum, deg, out):
        out[...] = osum[...] / jnp.maximum(deg[...], 1.0)

    return pl.pallas_call(
        body,
        grid=(grid,),
        in_specs=[
            pl.BlockSpec((bn, _OUT_W), lambda i: (i, 0)),
            pl.BlockSpec((bn, 1), lambda i: (i, 0)),
        ],
        out_specs=pl.BlockSpec((bn, _OUT_W), lambda i: (i, 0)),
        out_shape=jax.ShapeDtypeStruct((n, _OUT_W), jnp.float32),
    )


def kernel(x, edge_index, edge_attr, edge_vec, W1, b1, W2, b2, P0, P1, P2):
    n, ns = x.shape
    e = edge_index.shape[1]
    nv = P1.shape[1]
    src = edge_index[0]
    dst = edge_index[1]

    xsd = _make_gather(n, e, ns)(x, src, dst)

    evt = edge_vec.T
    ng = ns + 2 * nv
    pp = jnp.concatenate([P0, P1, P2], axis=1)
    zpad = jnp.zeros((ns, ng), jnp.float32)
    b1big = jnp.concatenate([
        jnp.concatenate([W1[:ns], zpad], axis=1),
        jnp.concatenate([W1[ns:2 * ns], pp], axis=1),
        jnp.concatenate([W1[2 * ns:], zpad], axis=1),
    ], axis=0)
    m = _make_messages(e, ns, nv)(
        edge_attr, xsd, evt, b1big, W2, b1.reshape(1, ns),
        b2.reshape(1, ng), _RCAT, _SCAT)

    z2 = jnp.zeros((n, _OUT_W // 8), jnp.float32)
    z1 = jnp.zeros((n,), jnp.float32)
    ones = jnp.ones((1000,), jnp.float32)
    osum, deg = _make_scatter(n, e)(m, dst, z2, z1, ones)

    return _make_divide(n)(osum, deg.reshape(n, 1))
